# baseline (speedup 1.0000x reference)
import functools

import jax
import jax.numpy as jnp
from jax import lax
from jax.experimental import pallas as pl
from jax.experimental.pallas import tpu as pltpu
from jax.experimental.pallas import tpu_sc as plsc

VOCAB = 100000
D_MODEL = 64
N_IDX = 16384
NUM_WORKERS = 32
B_PER_W = N_IDX // NUM_WORKERS
SQRT_D = float(D_MODEL) ** 0.5
LANES = 16
N_CHUNKS = B_PER_W // LANES
NBUF = 8
AHEAD = 4


def _fire_chunk(iv, w_hbm, dst, sem):
    for k in range(LANES):
        pltpu.async_copy(w_hbm.at[iv[k]], dst.at[k], sem)


def _drain_chunk(w_hbm, dst, sem):
    for k in range(LANES):
        pltpu.make_async_copy(w_hbm.at[0], dst.at[k], sem).wait()


def _body(x_hbm, w_hbm, out_hbm, idx_v, buf0, buf1, buf2, buf3,
          buf4, buf5, buf6, buf7, ov0, ov1,
          sem0, sem1, sem2, sem3, sem4, sem5, sem6, sem7, semo0, semo1):
    wid = lax.axis_index("s") * 2 + lax.axis_index("c")
    base = wid * B_PER_W

    pltpu.sync_copy(x_hbm.at[0, pl.ds(base, B_PER_W)], idx_v)
    buf = (buf0, buf1, buf2, buf3, buf4, buf5, buf6, buf7)
    sem = (sem0, sem1, sem2, sem3, sem4, sem5, sem6, sem7)
    ov = (ov0, ov1)
    semo = (semo0, semo1)

    for c0 in range(AHEAD):
        _fire_chunk(idx_v[pl.ds(c0 * LANES, LANES)], w_hbm, buf[c0], sem[c0])

    def quad(p, carry):
        for q in range(NBUF):
            c = NBUF * p + q
            b = q & 1
            iv = idx_v[pl.ds(c * LANES, LANES)]
            _drain_chunk(w_hbm, buf[q], sem[q])

            @pl.when(c + AHEAD < N_CHUNKS)
            def _fire():
                ivn = idx_v[pl.ds((c + AHEAD) * LANES, LANES)]
                _fire_chunk(ivn, w_hbm, buf[(q + AHEAD) % NBUF],
                            sem[(q + AHEAD) % NBUF])

            @pl.when(c >= 2)
            def _drain_out():
                pltpu.make_async_copy(
                    ov[b], out_hbm.at[pl.ds(0, LANES)], semo[b]).wait()

            for k in range(LANES):
                v0 = buf[q][k, pl.ds(0, LANES)]
                v1 = buf[q][k, pl.ds(LANES, LANES)]
                v2 = buf[q][k, pl.ds(2 * LANES, LANES)]
                v3 = buf[q][k, pl.ds(3 * LANES, LANES)]
                part = v0 * v0 + v1 * v1 + v2 * v2 + v3 * v3
                s = jnp.sum(part)

                i = lax.bitcast_convert_type(s, jnp.int32)
                i = jnp.int32(0x5F3759DF) - (i >> 1)
                y = lax.bitcast_convert_type(i, jnp.float32)
                h = s * jnp.float32(0.5)
                y = y * (jnp.float32(1.5) - h * y * y)
                y = y * (jnp.float32(1.5) - h * y * y)
                scale = jnp.minimum(jnp.float32(1.0), y) * jnp.float32(SQRT_D)
                scale = jnp.where(iv[k] != jnp.int32(0), scale,
                                  jnp.float32(0.0))
                sv = jnp.full((LANES,), scale, jnp.float32)

                ov[b][k, pl.ds(0, LANES)] = v0 * sv
                ov[b][k, pl.ds(LANES, LANES)] = v1 * sv
                ov[b][k, pl.ds(2 * LANES, LANES)] = v2 * sv
                ov[b][k, pl.ds(3 * LANES, LANES)] = v3 * sv

            pltpu.async_copy(ov[b], out_hbm.at[pl.ds(base + c * LANES, LANES)],
                             semo[b])
        return carry

    lax.fori_loop(0, N_CHUNKS // NBUF, quad, 0)

    pltpu.make_async_copy(ov[0], out_hbm.at[pl.ds(0, LANES)], semo[0]).wait()
    pltpu.make_async_copy(ov[1], out_hbm.at[pl.ds(0, LANES)], semo[1]).wait()


@jax.jit
def _sc_lookup(x, weight):
    mesh = plsc.VectorSubcoreMesh(core_axis_name="c", subcore_axis_name="s")
    return pl.kernel(
        _body,
        out_type=jax.ShapeDtypeStruct((N_IDX, D_MODEL), jnp.float32),
        mesh=mesh,
        scratch_types=[
            pltpu.VMEM((B_PER_W,), jnp.int32),
            pltpu.VMEM((LANES, D_MODEL), jnp.float32),
            pltpu.VMEM((LANES, D_MODEL), jnp.float32),
            pltpu.VMEM((LANES, D_MODEL), jnp.float32),
            pltpu.VMEM((LANES, D_MODEL), jnp.float32),
            pltpu.VMEM((LANES, D_MODEL), jnp.float32),
            pltpu.VMEM((LANES, D_MODEL), jnp.float32),
            pltpu.VMEM((LANES, D_MODEL), jnp.float32),
            pltpu.VMEM((LANES, D_MODEL), jnp.float32),
            pltpu.VMEM((LANES, D_MODEL), jnp.float32),
            pltpu.VMEM((LANES, D_MODEL), jnp.float32),
            pltpu.SemaphoreType.DMA,
            pltpu.SemaphoreType.DMA,
            pltpu.SemaphoreType.DMA,
            pltpu.SemaphoreType.DMA,
            pltpu.SemaphoreType.DMA,
            pltpu.SemaphoreType.DMA,
            pltpu.SemaphoreType.DMA,
            pltpu.SemaphoreType.DMA,
            pltpu.SemaphoreType.DMA,
            pltpu.SemaphoreType.DMA,
        ],
        compiler_params=pltpu.CompilerParams(
            needs_layout_passes=False, use_tc_tiling_on_sc=True),
    )(x, weight)


def kernel(x, weight):
    return _sc_lookup(x, weight)


# speedup vs baseline: 1.0398x; 1.0398x over previous
import functools

import jax
import jax.numpy as jnp
from jax import lax
from jax.experimental import pallas as pl
from jax.experimental.pallas import tpu as pltpu
from jax.experimental.pallas import tpu_sc as plsc

VOCAB = 100000
D_MODEL = 64
N_IDX = 16384
NUM_WORKERS = 32
B_PER_W = N_IDX // NUM_WORKERS
SQRT_D = float(D_MODEL) ** 0.5
LANES = 16
N_CHUNKS = B_PER_W // LANES
NBUF = 4


def _fire_chunk(iv, w_hbm, dst, sem):
    for k in range(LANES):
        pltpu.async_copy(w_hbm.at[iv[k]], dst.at[k], sem)


def _drain_chunk(w_hbm, dst, sem):
    for k in range(LANES):
        pltpu.make_async_copy(w_hbm.at[0], dst.at[k], sem).wait()


def _body(x_hbm, w_hbm, out_hbm, idx_v, buf0, buf1, buf2, buf3, ov0, ov1,
          sem0, sem1, sem2, sem3, semo0, semo1):
    wid = lax.axis_index("s") * 2 + lax.axis_index("c")
    base = wid * B_PER_W

    pltpu.sync_copy(x_hbm.at[0, pl.ds(base, B_PER_W)], idx_v)
    buf = (buf0, buf1, buf2, buf3)
    sem = (sem0, sem1, sem2, sem3)
    ov = (ov0, ov1)
    semo = (semo0, semo1)

    _fire_chunk(idx_v[pl.ds(0, LANES)], w_hbm, buf0, sem0)
    _fire_chunk(idx_v[pl.ds(LANES, LANES)], w_hbm, buf1, sem1)

    def quad(p, carry):
        for q in range(NBUF):
            c = NBUF * p + q
            b = q & 1
            iv = idx_v[pl.ds(c * LANES, LANES)]
            _drain_chunk(w_hbm, buf[q], sem[q])

            @pl.when(c + 2 < N_CHUNKS)
            def _fire():
                ivn = idx_v[pl.ds((c + 2) * LANES, LANES)]
                _fire_chunk(ivn, w_hbm, buf[(q + 2) & 3], sem[(q + 2) & 3])

            @pl.when(c >= 2)
            def _drain_out():
                pltpu.make_async_copy(
                    ov[b], out_hbm.at[pl.ds(0, LANES)], semo[b]).wait()

            for k in range(LANES):
                v0 = buf[q][k, pl.ds(0, LANES)]
                v1 = buf[q][k, pl.ds(LANES, LANES)]
                v2 = buf[q][k, pl.ds(2 * LANES, LANES)]
                v3 = buf[q][k, pl.ds(3 * LANES, LANES)]
                part = v0 * v0 + v1 * v1 + v2 * v2 + v3 * v3
                s = jnp.sum(part)

                i = lax.bitcast_convert_type(s, jnp.int32)
                i = jnp.int32(0x5F3759DF) - (i >> 1)
                y = lax.bitcast_convert_type(i, jnp.float32)
                h = s * jnp.float32(0.5)
                y = y * (jnp.float32(1.5) - h * y * y)
                y = y * (jnp.float32(1.5) - h * y * y)
                scale = jnp.minimum(jnp.float32(1.0), y) * jnp.float32(SQRT_D)
                scale = jnp.where(iv[k] != jnp.int32(0), scale,
                                  jnp.float32(0.0))
                sv = jnp.full((LANES,), scale, jnp.float32)

                ov[b][k, pl.ds(0, LANES)] = v0 * sv
                ov[b][k, pl.ds(LANES, LANES)] = v1 * sv
                ov[b][k, pl.ds(2 * LANES, LANES)] = v2 * sv
                ov[b][k, pl.ds(3 * LANES, LANES)] = v3 * sv

            pltpu.async_copy(ov[b], out_hbm.at[pl.ds(base + c * LANES, LANES)],
                             semo[b])
        return carry

    lax.fori_loop(0, N_CHUNKS // NBUF, quad, 0)

    pltpu.make_async_copy(ov[0], out_hbm.at[pl.ds(0, LANES)], semo[0]).wait()
    pltpu.make_async_copy(ov[1], out_hbm.at[pl.ds(0, LANES)], semo[1]).wait()


@jax.jit
def _sc_lookup(x, weight):
    mesh = plsc.VectorSubcoreMesh(core_axis_name="c", subcore_axis_name="s")
    return pl.kernel(
        _body,
        out_type=jax.ShapeDtypeStruct((N_IDX, D_MODEL), jnp.float32),
        mesh=mesh,
        scratch_types=[
            pltpu.VMEM((B_PER_W,), jnp.int32),
            pltpu.VMEM((LANES, D_MODEL), jnp.float32),
            pltpu.VMEM((LANES, D_MODEL), jnp.float32),
            pltpu.VMEM((LANES, D_MODEL), jnp.float32),
            pltpu.VMEM((LANES, D_MODEL), jnp.float32),
            pltpu.VMEM((LANES, D_MODEL), jnp.float32),
            pltpu.VMEM((LANES, D_MODEL), jnp.float32),
            pltpu.SemaphoreType.DMA,
            pltpu.SemaphoreType.DMA,
            pltpu.SemaphoreType.DMA,
            pltpu.SemaphoreType.DMA,
            pltpu.SemaphoreType.DMA,
            pltpu.SemaphoreType.DMA,
        ],
        compiler_params=pltpu.CompilerParams(
            needs_layout_passes=False, use_tc_tiling_on_sc=True),
    )(x, weight)


def kernel(x, weight):
    return _sc_lookup(x, weight)


# speedup vs baseline: 1.0931x; 1.0512x over previous
import functools

import jax
import jax.numpy as jnp
from jax import lax
from jax.experimental import pallas as pl
from jax.experimental.pallas import tpu as pltpu
from jax.experimental.pallas import tpu_sc as plsc

VOCAB = 100000
D_MODEL = 64
N_IDX = 16384
NUM_WORKERS = 32
B_PER_W = N_IDX // NUM_WORKERS
SQRT_D = float(D_MODEL) ** 0.5
LANES = 16
N_CHUNKS = B_PER_W // LANES
NBUF = 4


def _fire_chunk(iv, w_hbm, dst, sem):
    for k in range(LANES):
        pltpu.async_copy(w_hbm.at[iv[k]], dst.at[k], sem)


def _drain_chunk(w_hbm, dst, sem):
    for k in range(LANES):
        pltpu.make_async_copy(w_hbm.at[0], dst.at[k], sem).wait()


def _body(x_hbm, w_hbm, out_hbm, idx_v, buf0, buf1, buf2, buf3, ov0, ov1,
          sem0, sem1, sem2, sem3, semo0, semo1):
    wid = lax.axis_index("s") * 2 + lax.axis_index("c")
    base = wid * B_PER_W

    pltpu.sync_copy(x_hbm.at[0, pl.ds(base, B_PER_W)], idx_v)
    buf = (buf0, buf1, buf2, buf3)
    sem = (sem0, sem1, sem2, sem3)
    ov = (ov0, ov1)
    semo = (semo0, semo1)

    _fire_chunk(idx_v[pl.ds(0, LANES)], w_hbm, buf0, sem0)
    _fire_chunk(idx_v[pl.ds(LANES, LANES)], w_hbm, buf1, sem1)
    _fire_chunk(idx_v[pl.ds(2 * LANES, LANES)], w_hbm, buf2, sem2)

    def quad(p, carry):
        for q in range(NBUF):
            c = NBUF * p + q
            b = q & 1
            iv = idx_v[pl.ds(c * LANES, LANES)]
            _drain_chunk(w_hbm, buf[q], sem[q])

            @pl.when(c + 3 < N_CHUNKS)
            def _fire():
                ivn = idx_v[pl.ds((c + 3) * LANES, LANES)]
                _fire_chunk(ivn, w_hbm, buf[(q + 3) & 3], sem[(q + 3) & 3])

            @pl.when(c >= 2)
            def _drain_out():
                pltpu.make_async_copy(
                    ov[b], out_hbm.at[pl.ds(0, LANES)], semo[b]).wait()

            for k in range(LANES):
                v0 = buf[q][k, pl.ds(0, LANES)]
                v1 = buf[q][k, pl.ds(LANES, LANES)]
                v2 = buf[q][k, pl.ds(2 * LANES, LANES)]
                v3 = buf[q][k, pl.ds(3 * LANES, LANES)]
                part = v0 * v0 + v1 * v1 + v2 * v2 + v3 * v3
                s = jnp.sum(part)

                i = lax.bitcast_convert_type(s, jnp.int32)
                i = jnp.int32(0x5F3759DF) - (i >> 1)
                y = lax.bitcast_convert_type(i, jnp.float32)
                h = s * jnp.float32(0.5)
                y = y * (jnp.float32(1.5) - h * y * y)
                y = y * (jnp.float32(1.5) - h * y * y)
                scale = jnp.minimum(jnp.float32(1.0), y) * jnp.float32(SQRT_D)
                scale = jnp.where(iv[k] != jnp.int32(0), scale,
                                  jnp.float32(0.0))
                sv = jnp.full((LANES,), scale, jnp.float32)

                ov[b][k, pl.ds(0, LANES)] = v0 * sv
                ov[b][k, pl.ds(LANES, LANES)] = v1 * sv
                ov[b][k, pl.ds(2 * LANES, LANES)] = v2 * sv
                ov[b][k, pl.ds(3 * LANES, LANES)] = v3 * sv

            pltpu.async_copy(ov[b], out_hbm.at[pl.ds(base + c * LANES, LANES)],
                             semo[b])
        return carry

    lax.fori_loop(0, N_CHUNKS // NBUF, quad, 0)

    pltpu.make_async_copy(ov[0], out_hbm.at[pl.ds(0, LANES)], semo[0]).wait()
    pltpu.make_async_copy(ov[1], out_hbm.at[pl.ds(0, LANES)], semo[1]).wait()


@jax.jit
def _sc_lookup(x, weight):
    mesh = plsc.VectorSubcoreMesh(core_axis_name="c", subcore_axis_name="s")
    return pl.kernel(
        _body,
        out_type=jax.ShapeDtypeStruct((N_IDX, D_MODEL), jnp.float32),
        mesh=mesh,
        scratch_types=[
            pltpu.VMEM((B_PER_W,), jnp.int32),
            pltpu.VMEM((LANES, D_MODEL), jnp.float32),
            pltpu.VMEM((LANES, D_MODEL), jnp.float32),
            pltpu.VMEM((LANES, D_MODEL), jnp.float32),
            pltpu.VMEM((LANES, D_MODEL), jnp.float32),
            pltpu.VMEM((LANES, D_MODEL), jnp.float32),
            pltpu.VMEM((LANES, D_MODEL), jnp.float32),
            pltpu.SemaphoreType.DMA,
            pltpu.SemaphoreType.DMA,
            pltpu.SemaphoreType.DMA,
            pltpu.SemaphoreType.DMA,
            pltpu.SemaphoreType.DMA,
            pltpu.SemaphoreType.DMA,
        ],
        compiler_params=pltpu.CompilerParams(
            needs_layout_passes=False, use_tc_tiling_on_sc=True),
    )(x, weight)


def kernel(x, weight):
    return _sc_lookup(x, weight)


# speedup vs baseline: 1.0997x; 1.0061x over previous
import functools

import jax
import jax.numpy as jnp
from jax import lax
from jax.experimental import pallas as pl
from jax.experimental.pallas import tpu as pltpu
from jax.experimental.pallas import tpu_sc as plsc

VOCAB = 100000
D_MODEL = 64
N_IDX = 16384
NUM_WORKERS = 32
B_PER_W = N_IDX // NUM_WORKERS
SQRT_D = float(D_MODEL) ** 0.5
LANES = 16
N_CHUNKS = B_PER_W // LANES
NBUF = 4


def _fire_chunk(iv, w_hbm, dst, sem):
    for k in range(LANES):
        pltpu.async_copy(w_hbm.at[iv[k]], dst.at[k], sem)


def _drain_chunk(w_hbm, dst, sem):
    for k in range(LANES):
        pltpu.make_async_copy(w_hbm.at[0], dst.at[k], sem).wait()


def _body(x_hbm, w_hbm, out_hbm, idx_v, buf0, buf1, buf2, buf3, ov0, ov1,
          sem0, sem1, sem2, sem3, semo0, semo1):
    wid = lax.axis_index("s") * 2 + lax.axis_index("c")
    base = wid * B_PER_W

    pltpu.sync_copy(x_hbm.at[0, pl.ds(base, B_PER_W)], idx_v)
    buf = (buf0, buf1, buf2, buf3)
    sem = (sem0, sem1, sem2, sem3)
    ov = (ov0, ov1)
    semo = (semo0, semo1)

    _fire_chunk(idx_v[pl.ds(0, LANES)], w_hbm, buf0, sem0)
    _fire_chunk(idx_v[pl.ds(LANES, LANES)], w_hbm, buf1, sem1)
    _fire_chunk(idx_v[pl.ds(2 * LANES, LANES)], w_hbm, buf2, sem2)
    _fire_chunk(idx_v[pl.ds(3 * LANES, LANES)], w_hbm, buf3, sem3)

    def quad(p, carry):
        for q in range(NBUF):
            c = NBUF * p + q
            b = q & 1
            iv = idx_v[pl.ds(c * LANES, LANES)]
            _drain_chunk(w_hbm, buf[q], sem[q])

            @pl.when(c >= 2)
            def _drain_out():
                pltpu.make_async_copy(
                    ov[b], out_hbm.at[pl.ds(0, LANES)], semo[b]).wait()

            for k in range(LANES):
                v0 = buf[q][k, pl.ds(0, LANES)]
                v1 = buf[q][k, pl.ds(LANES, LANES)]
                v2 = buf[q][k, pl.ds(2 * LANES, LANES)]
                v3 = buf[q][k, pl.ds(3 * LANES, LANES)]
                part = v0 * v0 + v1 * v1 + v2 * v2 + v3 * v3
                s = jnp.sum(part)

                i = lax.bitcast_convert_type(s, jnp.int32)
                i = jnp.int32(0x5F3759DF) - (i >> 1)
                y = lax.bitcast_convert_type(i, jnp.float32)
                h = s * jnp.float32(0.5)
                y = y * (jnp.float32(1.5) - h * y * y)
                y = y * (jnp.float32(1.5) - h * y * y)
                scale = jnp.minimum(jnp.float32(1.0), y) * jnp.float32(SQRT_D)
                scale = jnp.where(iv[k] != jnp.int32(0), scale,
                                  jnp.float32(0.0))
                sv = jnp.full((LANES,), scale, jnp.float32)

                ov[b][k, pl.ds(0, LANES)] = v0 * sv
                ov[b][k, pl.ds(LANES, LANES)] = v1 * sv
                ov[b][k, pl.ds(2 * LANES, LANES)] = v2 * sv
                ov[b][k, pl.ds(3 * LANES, LANES)] = v3 * sv

            pltpu.async_copy(ov[b], out_hbm.at[pl.ds(base + c * LANES, LANES)],
                             semo[b])

            @pl.when(c + NBUF < N_CHUNKS)
            def _fire():
                ivn = idx_v[pl.ds((c + NBUF) * LANES, LANES)]
                _fire_chunk(ivn, w_hbm, buf[q], sem[q])
        return carry

    lax.fori_loop(0, N_CHUNKS // NBUF, quad, 0)

    pltpu.make_async_copy(ov[0], out_hbm.at[pl.ds(0, LANES)], semo[0]).wait()
    pltpu.make_async_copy(ov[1], out_hbm.at[pl.ds(0, LANES)], semo[1]).wait()


@jax.jit
def _sc_lookup(x, weight):
    mesh = plsc.VectorSubcoreMesh(core_axis_name="c", subcore_axis_name="s")
    return pl.kernel(
        _body,
        out_type=jax.ShapeDtypeStruct((N_IDX, D_MODEL), jnp.float32),
        mesh=mesh,
        scratch_types=[
            pltpu.VMEM((B_PER_W,), jnp.int32),
            pltpu.VMEM((LANES, D_MODEL), jnp.float32),
            pltpu.VMEM((LANES, D_MODEL), jnp.float32),
            pltpu.VMEM((LANES, D_MODEL), jnp.float32),
            pltpu.VMEM((LANES, D_MODEL), jnp.float32),
            pltpu.VMEM((LANES, D_MODEL), jnp.float32),
            pltpu.VMEM((LANES, D_MODEL), jnp.float32),
            pltpu.SemaphoreType.DMA,
            pltpu.SemaphoreType.DMA,
            pltpu.SemaphoreType.DMA,
            pltpu.SemaphoreType.DMA,
            pltpu.SemaphoreType.DMA,
            pltpu.SemaphoreType.DMA,
        ],
        compiler_params=pltpu.CompilerParams(
            needs_layout_passes=False, use_tc_tiling_on_sc=True),
    )(x, weight)


def kernel(x, weight):
    return _sc_lookup(x, weight)
